# baseline (speedup 1.0000x reference)
import functools

import jax
import jax.numpy as jnp
from jax import lax
from jax.experimental import pallas as pl
from jax.experimental.pallas import tpu as pltpu
from jax.experimental.pallas import tpu_sc as plsc

_EPS = 1e-5
_K = 20
_B = 2
_N = 2048
_CIN = 1216
_D = 256

_NC = 2
_NS = 16
_NW = _NC * _NS
_RPW = _N // _NW
_RCH = 4
_NCH = _RPW // _RCH
_IDXC = _RCH * _K

_NB1 = 8
_NBLK = _N // _NB1


def _leaky(v):
    return jnp.where(v > 0, v, 0.2 * v)


def _bdot(a, b, dn):
    return lax.dot_general(a.astype(jnp.bfloat16), b.astype(jnp.bfloat16),
                           dn, preferred_element_type=jnp.float32)


def _k12_body(x_ref, w1_ref, c1_ref, wn_ref, wd_ref, c2_ref, pt_ref, p_ref,
              a_ref, c_ref, idx_ref):
    xb = x_ref[0].astype(jnp.bfloat16)
    h = lax.dot_general(w1_ref[...], xb, (((1,), (0,)), ((), ())),
                        preferred_element_type=jnp.float32)
    h = _leaky(h + c1_ref[...])
    dn = (((0,), (1,)), ((), ()))
    h16 = h.astype(jnp.bfloat16)
    a_ref[...] = lax.dot_general(h16, wn_ref[...], dn,
                                 preferred_element_type=jnp.float32)
    c_ref[...] = lax.dot_general(h16, wd_ref[...], dn,
                                 preferred_element_type=jnp.float32) + c2_ref[...]

    a = pt_ref[0]
    pr = p_ref[0]
    pp_row = jnp.sum(a * a, axis=1, keepdims=True)
    pp_col = jnp.sum(pr * pr, axis=0, keepdims=True)
    prod = _bdot(a, pr, (((1,), (0,)), ((), ())))
    dist = (pp_row + (-2.0 * prod)) + pp_col
    bits = lax.bitcast_convert_type(jnp.maximum(dist, 0.0), jnp.int32)
    cols = lax.broadcasted_iota(jnp.int32, (_NBLK, _N), 1)
    key = (bits & jnp.int32(~0x7FF)) | cols
    outs = []
    for _ in range(_K):
        m = jnp.min(key, axis=1, keepdims=True)
        outs.append(m & jnp.int32(0x7FF))
        key = jnp.where(key == m, jnp.int32(0x7FFFFFFF), key)
    idx_ref[0] = jnp.concatenate(outs, axis=1)


def _k12_kw(b):
    return dict(
        grid=(_NB1,),
        in_specs=[
            pl.BlockSpec((1, _CIN, _NBLK), lambda j: (b, 0, j)),
            pl.BlockSpec((_D, _CIN), lambda j: (0, 0)),
            pl.BlockSpec((_D, 1), lambda j: (0, 0)),
            pl.BlockSpec((_D, _D), lambda j: (0, 0)),
            pl.BlockSpec((_D, _D), lambda j: (0, 0)),
            pl.BlockSpec((1, _D), lambda j: (0, 0)),
            pl.BlockSpec((1, _NBLK, 3), lambda j: (b, j, 0)),
            pl.BlockSpec((1, 3, _N), lambda j: (b, 0, 0)),
        ],
        out_specs=(
            pl.BlockSpec((_NBLK, _D), lambda j: (j, 0)),
            pl.BlockSpec((_NBLK, _D), lambda j: (j, 0)),
            pl.BlockSpec((1, _NBLK, _K), lambda j: (0, j, 0)),
        ),
        out_shape=(
            jax.ShapeDtypeStruct((_N, _D), jnp.float32),
            jax.ShapeDtypeStruct((_N, _D), jnp.float32),
            jax.ShapeDtypeStruct((1, _N, _K), jnp.int32),
        ),
    )


def _gmax_body(a_hbm, idx_hbm, out_hbm, idx_v, rows_v, outb_v, sem0, sem1):
    wid = lax.axis_index("s") * _NC + lax.axis_index("c")
    row0 = wid * _RPW
    pltpu.sync_copy(idx_hbm.at[wid], idx_v)
    sems = (sem0, sem1)

    def start(c, b):
        pltpu.async_copy(a_hbm.at[idx_v.at[c]], rows_v.at[b], sems[b])

    def wait(b):
        pltpu.make_async_copy(a_hbm.at[idx_v.at[0]], rows_v.at[b],
                              sems[b]).wait()

    start(0, 0)
    start(1, 1)

    def outer(i, carry):
        for b in (0, 1):
            c = 2 * i + b
            wait(b)

            def row_body(r, rc):
                base = r * _K
                for lg in range(_D // 16):
                    sl = pl.ds(lg * 16, 16)
                    acc = rows_v[b, base, sl]
                    for j in range(1, _K):
                        acc = jnp.maximum(acc, rows_v[b, base + j, sl])
                    outb_v[r, sl] = acc
                return rc

            lax.fori_loop(0, _RCH, row_body, 0)

            @pl.when(c + 2 < _NCH)
            def _():
                start(c + 2, b)

            ro = pl.multiple_of(row0 + c * _RCH, 4)
            pltpu.sync_copy(outb_v, out_hbm.at[pl.ds(ro, _RCH)])
        return carry

    lax.fori_loop(0, _NCH // 2, outer, 0)


@functools.cache
def _gmax_call():
    return functools.partial(
        pl.kernel,
        out_type=jax.ShapeDtypeStruct((_N, _D), jnp.float32),
        mesh=plsc.VectorSubcoreMesh(core_axis_name="c", subcore_axis_name="s",
                                    num_cores=_NC, num_subcores=_NS),
        cost_estimate=pl.CostEstimate(flops=11_000_000, transcendentals=0,
                                      bytes_accessed=45_000_000),
        scratch_types=[
            pltpu.VMEM((_NCH, _IDXC), jnp.int32),
            pltpu.VMEM((2, _IDXC, _D), jnp.float32),
            pltpu.VMEM((_RCH, _D), jnp.float32),
            pltpu.SemaphoreType.DMA,
            pltpu.SemaphoreType.DMA,
        ],
    )(_gmax_body)


def _k4_body(m_ref, c_ref, w3_ref, c3_ref, w4_ref, c4_ref, w5_ref, b5_ref,
             out_ref):
    xb = _leaky(m_ref[...] + c_ref[...])
    dn = (((1,), (1,)), ((), ()))
    h3 = _leaky(_bdot(xb, w3_ref[...], dn) + c3_ref[...])
    h4 = _leaky(_bdot(h3, w4_ref[...], dn) + c4_ref[...])
    ot = _bdot(w5_ref[...], h4, dn) + b5_ref[...]
    out_ref[...] = ot


_K4_KW = dict(
    grid=(_NB1,),
    in_specs=[
        pl.BlockSpec((_NBLK, _D), lambda j: (j, 0)),
        pl.BlockSpec((_NBLK, _D), lambda j: (j, 0)),
        pl.BlockSpec((_D, _D), lambda j: (0, 0)),
        pl.BlockSpec((1, _D), lambda j: (0, 0)),
        pl.BlockSpec((128, _D), lambda j: (0, 0)),
        pl.BlockSpec((1, 128), lambda j: (0, 0)),
        pl.BlockSpec((13, 128), lambda j: (0, 0)),
        pl.BlockSpec((13, 1), lambda j: (0, 0)),
    ],
    out_specs=pl.BlockSpec((13, _NBLK), lambda j: (0, j)),
    out_shape=jax.ShapeDtypeStruct((13, _N), jnp.float32),
)


def kernel(x, p, W1, b1, g1, be1, W2, b2, g2, be2, W3, b3, g3, be3,
           W4, b4, g4, be4, W5, b5):
    s = 1.0 / jnp.sqrt(1.0 + _EPS)
    s1 = g1 * s
    w1e = (W1 * s1[:, None]).astype(jnp.bfloat16)
    c1 = (b1 * s1 + be1).reshape(_D, 1)
    s2 = g2 * s
    wn = (W2[:, :_D] * s2[:, None]).astype(jnp.bfloat16)
    wd = ((W2[:, _D:] - W2[:, :_D]) * s2[:, None]).astype(jnp.bfloat16)
    c2 = (b2 * s2 + be2).reshape(1, _D)
    s3 = g3 * s
    w3e = W3 * s3[:, None]
    c3 = (b3 * s3 + be3).reshape(1, _D)
    s4 = g4 * s
    w4e = W4 * s4[:, None]
    c4 = (b4 * s4 + be4).reshape(1, 128)
    b5c = b5.reshape(13, 1)
    pt = jnp.transpose(p, (0, 2, 1))

    k4 = pl.pallas_call(_k4_body, **_K4_KW)
    outs = []
    parts = []
    for b in range(_B):
        k12 = pl.pallas_call(_k12_body, **_k12_kw(b))
        ab, cb, idxb = k12(x, w1e, c1, wn, wd, c2, pt, p)
        parts.append((ab, cb, idxb))
    for b in range(_B):
        ab, cb, idxb = parts[b]
        mb = _gmax_call()(ab, idxb.reshape(_NW, _NCH, _IDXC))
        outs.append(k4(mb, cb, w3e, c3, w4e, c4, W5, b5c))
    return jnp.stack(outs, axis=0)


# speedup vs baseline: 1.2936x; 1.2936x over previous
import functools

import jax
import jax.numpy as jnp
from jax import lax
from jax.experimental import pallas as pl
from jax.experimental.pallas import tpu as pltpu
from jax.experimental.pallas import tpu_sc as plsc

_EPS = 1e-5
_K = 20
_B = 2
_N = 2048
_CIN = 1216
_D = 256

_NC = 2
_NS = 16
_NW = _NC * _NS
_RPW = _N // _NW
_RCH = 4
_NCH = _RPW // _RCH
_IDXC = _RCH * _K

_NB1 = 8
_NBLK = _N // _NB1


def _leaky(v):
    return jnp.where(v > 0, v, 0.2 * v)


def _bdot(a, b, dn):
    return lax.dot_general(a.astype(jnp.bfloat16), b.astype(jnp.bfloat16),
                           dn, preferred_element_type=jnp.float32)


def _k12_body(x_ref, w1_ref, c1_ref, wn_ref, wd_ref, c2_ref, pt_ref, p_ref,
              a_ref, c_ref, idx_ref):
    xb = x_ref[0].astype(jnp.bfloat16)
    h = lax.dot_general(w1_ref[...], xb, (((1,), (0,)), ((), ())),
                        preferred_element_type=jnp.float32)
    h = _leaky(h + c1_ref[...])
    dn = (((0,), (1,)), ((), ()))
    h16 = h.astype(jnp.bfloat16)
    a_ref[...] = lax.dot_general(h16, wn_ref[...], dn,
                                 preferred_element_type=jnp.float32)
    c_ref[...] = lax.dot_general(h16, wd_ref[...], dn,
                                 preferred_element_type=jnp.float32) + c2_ref[...]

    a = pt_ref[0]
    pr = p_ref[0]
    pp_row = jnp.sum(a * a, axis=1, keepdims=True)
    pp_col = jnp.sum(pr * pr, axis=0, keepdims=True)
    prod = _bdot(a, pr, (((1,), (0,)), ((), ())))
    dist = (pp_row + (-2.0 * prod)) + pp_col
    bits = lax.bitcast_convert_type(jnp.maximum(dist, 0.0), jnp.int32)
    cols = lax.broadcasted_iota(jnp.int32, (_NBLK, _N), 1)
    key = (bits & jnp.int32(~0x7FF)) | cols
    maxi = jnp.int32(0x7FFFFFFF)
    cands = []
    for t in range(4):
        m = key[:, 0:128]
        for g in range(1, 16):
            m = jnp.minimum(m, key[:, g * 128:(g + 1) * 128])
        cands.append(m)
        if t < 3:
            mt = jnp.concatenate([m] * 16, axis=1)
            key = jnp.where(key == mt, maxi, key)
    cand = jnp.concatenate(cands, axis=1)
    outs = []
    for _ in range(_K):
        mm = jnp.min(cand, axis=1, keepdims=True)
        outs.append(mm & jnp.int32(0x7FF))
        cand = jnp.where(cand == mm, maxi, cand)
    idx_ref[0] = jnp.concatenate(outs, axis=1)


def _k12_kw(b):
    return dict(
        grid=(_NB1,),
        in_specs=[
            pl.BlockSpec((1, _CIN, _NBLK), lambda j: (b, 0, j)),
            pl.BlockSpec((_D, _CIN), lambda j: (0, 0)),
            pl.BlockSpec((_D, 1), lambda j: (0, 0)),
            pl.BlockSpec((_D, _D), lambda j: (0, 0)),
            pl.BlockSpec((_D, _D), lambda j: (0, 0)),
            pl.BlockSpec((1, _D), lambda j: (0, 0)),
            pl.BlockSpec((1, _NBLK, 3), lambda j: (b, j, 0)),
            pl.BlockSpec((1, 3, _N), lambda j: (b, 0, 0)),
        ],
        out_specs=(
            pl.BlockSpec((_NBLK, _D), lambda j: (j, 0)),
            pl.BlockSpec((_NBLK, _D), lambda j: (j, 0)),
            pl.BlockSpec((1, _NBLK, _K), lambda j: (0, j, 0)),
        ),
        out_shape=(
            jax.ShapeDtypeStruct((_N, _D), jnp.float32),
            jax.ShapeDtypeStruct((_N, _D), jnp.float32),
            jax.ShapeDtypeStruct((1, _N, _K), jnp.int32),
        ),
    )


def _gmax_body(a_hbm, idx_hbm, out_hbm, idx_v, rows_v, outb_v, sem0, sem1):
    wid = lax.axis_index("s") * _NC + lax.axis_index("c")
    row0 = wid * _RPW
    pltpu.sync_copy(idx_hbm.at[wid], idx_v)
    sems = (sem0, sem1)

    def start(c, b):
        pltpu.async_copy(a_hbm.at[idx_v.at[c]], rows_v.at[b], sems[b])

    def wait(b):
        pltpu.make_async_copy(a_hbm.at[idx_v.at[0]], rows_v.at[b],
                              sems[b]).wait()

    start(0, 0)
    start(1, 1)

    def outer(i, carry):
        for b in (0, 1):
            c = 2 * i + b
            wait(b)

            def row_body(r, rc):
                base = r * _K
                for lg in range(_D // 16):
                    sl = pl.ds(lg * 16, 16)
                    acc = rows_v[b, base, sl]
                    for j in range(1, _K):
                        acc = jnp.maximum(acc, rows_v[b, base + j, sl])
                    outb_v[r, sl] = acc
                return rc

            lax.fori_loop(0, _RCH, row_body, 0)

            @pl.when(c + 2 < _NCH)
            def _():
                start(c + 2, b)

            ro = pl.multiple_of(row0 + c * _RCH, 4)
            pltpu.sync_copy(outb_v, out_hbm.at[pl.ds(ro, _RCH)])
        return carry

    lax.fori_loop(0, _NCH // 2, outer, 0)


@functools.cache
def _gmax_call():
    return functools.partial(
        pl.kernel,
        out_type=jax.ShapeDtypeStruct((_N, _D), jnp.float32),
        mesh=plsc.VectorSubcoreMesh(core_axis_name="c", subcore_axis_name="s",
                                    num_cores=_NC, num_subcores=_NS),
        cost_estimate=pl.CostEstimate(flops=11_000_000, transcendentals=0,
                                      bytes_accessed=45_000_000),
        scratch_types=[
            pltpu.VMEM((_NCH, _IDXC), jnp.int32),
            pltpu.VMEM((2, _IDXC, _D), jnp.float32),
            pltpu.VMEM((_RCH, _D), jnp.float32),
            pltpu.SemaphoreType.DMA,
            pltpu.SemaphoreType.DMA,
        ],
    )(_gmax_body)


def _k4_body(m_ref, c_ref, w3_ref, c3_ref, w4_ref, c4_ref, w5_ref, b5_ref,
             out_ref):
    xb = _leaky(m_ref[...] + c_ref[...])
    dn = (((1,), (1,)), ((), ()))
    h3 = _leaky(_bdot(xb, w3_ref[...], dn) + c3_ref[...])
    h4 = _leaky(_bdot(h3, w4_ref[...], dn) + c4_ref[...])
    ot = _bdot(w5_ref[...], h4, dn) + b5_ref[...]
    out_ref[...] = ot


_K4_KW = dict(
    grid=(_NB1,),
    in_specs=[
        pl.BlockSpec((_NBLK, _D), lambda j: (j, 0)),
        pl.BlockSpec((_NBLK, _D), lambda j: (j, 0)),
        pl.BlockSpec((_D, _D), lambda j: (0, 0)),
        pl.BlockSpec((1, _D), lambda j: (0, 0)),
        pl.BlockSpec((128, _D), lambda j: (0, 0)),
        pl.BlockSpec((1, 128), lambda j: (0, 0)),
        pl.BlockSpec((13, 128), lambda j: (0, 0)),
        pl.BlockSpec((13, 1), lambda j: (0, 0)),
    ],
    out_specs=pl.BlockSpec((13, _NBLK), lambda j: (0, j)),
    out_shape=jax.ShapeDtypeStruct((13, _N), jnp.float32),
)


def kernel(x, p, W1, b1, g1, be1, W2, b2, g2, be2, W3, b3, g3, be3,
           W4, b4, g4, be4, W5, b5):
    s = 1.0 / jnp.sqrt(1.0 + _EPS)
    s1 = g1 * s
    w1e = (W1 * s1[:, None]).astype(jnp.bfloat16)
    c1 = (b1 * s1 + be1).reshape(_D, 1)
    s2 = g2 * s
    wn = (W2[:, :_D] * s2[:, None]).astype(jnp.bfloat16)
    wd = ((W2[:, _D:] - W2[:, :_D]) * s2[:, None]).astype(jnp.bfloat16)
    c2 = (b2 * s2 + be2).reshape(1, _D)
    s3 = g3 * s
    w3e = W3 * s3[:, None]
    c3 = (b3 * s3 + be3).reshape(1, _D)
    s4 = g4 * s
    w4e = W4 * s4[:, None]
    c4 = (b4 * s4 + be4).reshape(1, 128)
    b5c = b5.reshape(13, 1)
    pt = jnp.transpose(p, (0, 2, 1))

    k4 = pl.pallas_call(_k4_body, **_K4_KW)
    outs = []
    parts = []
    for b in range(_B):
        k12 = pl.pallas_call(_k12_body, **_k12_kw(b))
        ab, cb, idxb = k12(x, w1e, c1, wn, wd, c2, pt, p)
        parts.append((ab, cb, idxb))
    for b in range(_B):
        ab, cb, idxb = parts[b]
        mb = _gmax_call()(ab, idxb.reshape(_NW, _NCH, _IDXC))
        outs.append(k4(mb, cb, w3e, c3, w4e, c4, W5, b5c))
    return jnp.stack(outs, axis=0)
